# baseline (speedup 1.0000x reference)
import jax
import jax.numpy as jnp
from jax import lax
from jax.experimental import pallas as pl
from jax.experimental.pallas import tpu as pltpu
from jax.experimental.pallas import tpu_sc as plsc

_B = 8
_H = 32
_W = 32
_D = 256
_HW = _H * _W
_NCH = 2 * _D
_RPW = 32


def _sc_body(col_hbm, row_hbm, out_hbm, patt_v, colbuf_v, rbuf_v, sem):
    cid = lax.axis_index("c")
    sid = lax.axis_index("s")
    wid = cid * 16 + sid

    pltpu.sync_copy(col_hbm, colbuf_v)
    pltpu.sync_copy(row_hbm.at[wid], rbuf_v)
    vregs = [rbuf_v[pl.ds(g * 16, 16)] for g in range(_D // 16)]
    for j in range(_RPW):
        for g in range(_D // 16):
            patt_v[j, pl.ds(g * 16, 16)] = colbuf_v[j, pl.ds(g * 16, 16)]
        for g in range(_D // 16):
            patt_v[j, pl.ds(_D + g * 16, 16)] = vregs[g]

    copies = [
        pltpu.make_async_copy(
            patt_v, out_hbm.at[b, pl.ds(wid * _RPW, _RPW), :], sem
        )
        for b in range(_B)
    ]
    for c in copies:
        c.start()
    for c in copies:
        c.wait()


def kernel(x, row_embed, col_embed):
    b = x.shape[0]
    h, w = x.shape[-2], x.shape[-1]
    d = col_embed.shape[-1]
    col = col_embed[:w]
    row = row_embed[:h]
    mesh = plsc.VectorSubcoreMesh(core_axis_name="c", subcore_axis_name="s")
    run = pl.kernel(
        _sc_body,
        mesh=mesh,
        out_type=jax.ShapeDtypeStruct((b, h * w, 2 * d), jnp.float32),
        scratch_types=[
            pltpu.VMEM((_RPW, _NCH), jnp.float32),
            pltpu.VMEM((_W, _D), jnp.float32),
            pltpu.VMEM((_D,), jnp.float32),
            pltpu.SemaphoreType.DMA,
        ],
    )
    out = run(col, row)
    return out.reshape(b, h, w, 2 * d).transpose(0, 3, 1, 2)


# speedup vs baseline: 3.0432x; 3.0432x over previous
import jax
import jax.numpy as jnp
from jax.experimental import pallas as pl
from jax.experimental.pallas import tpu as pltpu

_B = 8


def _pos_kernel(col_ref, row_ref, out_ref, patt_ref, sems):
    col = col_ref[...]
    row = row_ref[...]
    h, w = row.shape[0], col.shape[0]
    d = col.shape[1]
    colpat = jnp.broadcast_to(col[None], (h, w, d)).reshape(h * w, d)
    rowpat = jnp.broadcast_to(row[:, None, :], (h, w, d)).reshape(h * w, d)
    patt = jnp.concatenate([colpat, rowpat], axis=1)
    patt_ref[...] = jnp.broadcast_to(patt[None], (2,) + patt.shape)
    copies = [
        pltpu.make_async_copy(patt_ref.at[b % 2], out_ref.at[b], sems.at[b])
        for b in range(_B)
    ]
    for c in copies:
        c.start()
    for c in copies:
        c.wait()


def kernel(x, row_embed, col_embed):
    b = x.shape[0]
    h, w = x.shape[-2], x.shape[-1]
    d = col_embed.shape[-1]
    col = col_embed[:w]
    row = row_embed[:h]
    out = pl.pallas_call(
        _pos_kernel,
        in_specs=[
            pl.BlockSpec(memory_space=pltpu.VMEM),
            pl.BlockSpec(memory_space=pltpu.VMEM),
        ],
        out_specs=pl.BlockSpec(memory_space=pl.MemorySpace.ANY),
        out_shape=jax.ShapeDtypeStruct((b, h * w, 2 * d), jnp.float32),
        scratch_shapes=[
            pltpu.VMEM((2, h * w, 2 * d), jnp.float32),
            pltpu.SemaphoreType.DMA((b,)),
        ],
    )(col, row)
    return out.reshape(b, h, w, 2 * d).transpose(0, 3, 1, 2)


# speedup vs baseline: 3.1223x; 1.0260x over previous
import jax
import jax.numpy as jnp
from jax.experimental import pallas as pl
from jax.experimental.pallas import tpu as pltpu

_B = 8


def _pos_kernel(col_ref, row_ref, out_ref, patt_ref, sems):
    col = col_ref[...]
    row = row_ref[...]
    h, w = row.shape[0], col.shape[0]
    d = col.shape[1]
    colpat = jnp.broadcast_to(col[None], (h, w, d)).reshape(h * w, d)
    rowpat = jnp.broadcast_to(row[:, None, :], (h, w, d)).reshape(h * w, d)
    patt_ref[...] = jnp.concatenate([colpat, rowpat], axis=1)
    copies = [
        pltpu.make_async_copy(patt_ref, out_ref.at[b], sems.at[b])
        for b in range(_B)
    ]
    for c in copies:
        c.start()
    for c in copies:
        c.wait()


def kernel(x, row_embed, col_embed):
    b = x.shape[0]
    h, w = x.shape[-2], x.shape[-1]
    d = col_embed.shape[-1]
    col = col_embed[:w]
    row = row_embed[:h]
    out = pl.pallas_call(
        _pos_kernel,
        in_specs=[
            pl.BlockSpec(memory_space=pltpu.VMEM),
            pl.BlockSpec(memory_space=pltpu.VMEM),
        ],
        out_specs=pl.BlockSpec(memory_space=pl.MemorySpace.ANY),
        out_shape=jax.ShapeDtypeStruct((b, h * w, 2 * d), jnp.float32),
        scratch_shapes=[
            pltpu.VMEM((h * w, 2 * d), jnp.float32),
            pltpu.SemaphoreType.DMA((b,)),
        ],
    )(col, row)
    return out.reshape(b, h, w, 2 * d).transpose(0, 3, 1, 2)
